# baseline (speedup 1.0000x reference)
import functools

import jax
import jax.numpy as jnp
from jax import lax
from jax.experimental import pallas as pl
from jax.experimental.pallas import tpu as pltpu
from jax.experimental.pallas import tpu_sc as plsc

N = 16384
NNZ = 268435
D_IN = 1024
D_OUT = 32

NC = 2
NS = 16
NW = NC * NS
CH = 128
CHUNKS = -(-NNZ // (NW * CH))
PER_W = CHUNKS * CH
NNZ_PAD = NW * PER_W
RPS = N // NS


def _matmul(b, W, bias2d):
    m_blk = 2048

    def body(b_ref, w_ref, bias_ref, o_ref):
        o_ref[...] = (
            jnp.dot(b_ref[...], w_ref[...], preferred_element_type=jnp.float32)
            + bias_ref[...]
        )

    return pl.pallas_call(
        body,
        grid=(N // m_blk,),
        in_specs=[
            pl.BlockSpec((m_blk, D_IN), lambda i: (i, 0)),
            pl.BlockSpec((D_IN, D_OUT), lambda i: (0, 0)),
            pl.BlockSpec((1, D_OUT), lambda i: (0, 0)),
        ],
        out_specs=pl.BlockSpec((m_blk, D_OUT), lambda i: (i, 0)),
        out_shape=jax.ShapeDtypeStruct((N, D_OUT), jnp.float32),
    )(b, W, bias2d)


def _combine(partials):
    m_blk = 2048

    def body(p0_ref, p1_ref, o_ref):
        o_ref[...] = p0_ref[...] + p1_ref[...]

    return pl.pallas_call(
        body,
        grid=(N // m_blk,),
        in_specs=[
            pl.BlockSpec((m_blk, D_OUT), lambda i: (i, 0)),
            pl.BlockSpec((m_blk, D_OUT), lambda i: (i + N // m_blk, 0)),
        ],
        out_specs=pl.BlockSpec((m_blk, D_OUT), lambda i: (i, 0)),
        out_shape=jax.ShapeDtypeStruct((N, D_OUT), jnp.float32),
    )(partials, partials)


NB = 3


def _sc_spmm(cols2d, rows2d, vals2d, b_fc):
    mesh = plsc.VectorSubcoreMesh(core_axis_name="c", subcore_axis_name="s")

    @functools.partial(
        pl.kernel,
        out_type=jax.ShapeDtypeStruct((NC * N, D_OUT), jnp.float32),
        mesh=mesh,
        compiler_params=pltpu.CompilerParams(use_tc_tiling_on_sc=False),
        scratch_types=[
            pltpu.VMEM((CHUNKS, CH), jnp.int32),
            pltpu.VMEM((CHUNKS, CH), jnp.int32),
            pltpu.VMEM((CHUNKS, CH), jnp.float32),
            pltpu.VMEM((CH, D_OUT), jnp.float32),
            pltpu.VMEM((CH, D_OUT), jnp.float32),
            pltpu.VMEM((CH, D_OUT), jnp.float32),
            pltpu.VMEM_SHARED((N, D_OUT), jnp.float32),
            pltpu.VMEM_SHARED((N, D_OUT), jnp.float32),
            pltpu.SemaphoreType.DMA,
            pltpu.SemaphoreType.DMA,
            pltpu.SemaphoreType.DMA,
            pltpu.SemaphoreType.DMA,
            pltpu.SemaphoreType.DMA,
            pltpu.SemaphoreType.DMA,
            pltpu.SemaphoreType.DMA,
            pltpu.SemaphoreType.DMA,
            pltpu.SemaphoreType.DMA,
            pltpu.SemaphoreType.DMA,
        ],
    )
    def k(cols_hbm, rows_hbm, vals_hbm, bfc_hbm, out_hbm, col_all, row_all,
          val_all, g0, g1, g2, acc_sh, bfc_sh, sem_c, sem_r, sem_v, sem_b,
          sg0, sg1, sg2, ss0, ss1, ss2):
        gat = (g0, g1, g2)
        semg = (sg0, sg1, sg2)
        sems = (ss0, ss1, ss2)
        cid = lax.axis_index("c")
        sid = lax.axis_index("s")
        wid = cid * NS + sid

        c_copy = pltpu.async_copy(
            cols_hbm.at[pl.ds(wid * CHUNKS, CHUNKS)], col_all, sem_c)
        r_copy = pltpu.async_copy(
            rows_hbm.at[pl.ds(wid * CHUNKS, CHUNKS)], row_all, sem_r)
        v_copy = pltpu.async_copy(
            vals_hbm.at[pl.ds(wid * CHUNKS, CHUNKS)], val_all, sem_v)
        b_copy = pltpu.async_copy(
            bfc_hbm.at[pl.ds(sid * RPS, RPS)],
            bfc_sh.at[pl.ds(sid * RPS, RPS)], sem_b)

        def zfill(i, _):
            g0[i, pl.ds(0, 16)] = jnp.zeros((16,), jnp.float32)
            g0[i, pl.ds(16, 16)] = jnp.zeros((16,), jnp.float32)
            return 0

        lax.fori_loop(0, CH, zfill, 0, unroll=4)

        def zcopy(t, _):
            off = pl.multiple_of(sid * RPS + t * CH, CH)
            pltpu.sync_copy(g0, acc_sh.at[pl.ds(off, CH)])
            return 0

        lax.fori_loop(0, RPS // CH, zcopy, 0)
        c_copy.wait()
        r_copy.wait()
        v_copy.wait()
        b_copy.wait()
        plsc.subcore_barrier()

        for b in range(NB - 1):
            pltpu.async_copy(bfc_sh.at[col_all.at[b]], gat[b], semg[b])

        def scale(gbuf, t, j0):
            v16 = val_all[t, pl.ds(j0 * 16, 16)]
            for r in range(16):
                j = j0 * 16 + r
                gbuf[j, pl.ds(0, 16)] = gbuf[j, pl.ds(0, 16)] * v16[r]
                gbuf[j, pl.ds(16, 16)] = gbuf[j, pl.ds(16, 16)] * v16[r]

        def outer(g, _):
            for b in range(NB):
                t = g * NB + b
                pltpu.make_async_copy(
                    bfc_sh.at[col_all.at[t]], gat[b], semg[b]).wait()
                u = t + NB - 1
                bb = (b + NB - 1) % NB

                @pl.when(u < CHUNKS)
                def _():
                    @pl.when(t >= 1)
                    def _():
                        pltpu.make_async_copy(
                            gat[bb], acc_sh.at[row_all.at[t - 1]],
                            sems[bb]).wait()

                    pltpu.async_copy(
                        bfc_sh.at[col_all.at[u]], gat[bb], semg[bb])

                def sbody(j0, _):
                    scale(gat[b], t, j0)
                    return 0

                lax.fori_loop(0, CH // 16, sbody, 0)
                pltpu.async_copy(
                    gat[b], acc_sh.at[row_all.at[t]], sems[b], add=True)
            return 0

        lax.fori_loop(0, CHUNKS // NB, outer, 0)

        for b in range(NB):
            pltpu.make_async_copy(
                gat[b], acc_sh.at[row_all.at[CHUNKS - NB + b]],
                sems[b]).wait()
        plsc.subcore_barrier()

        pltpu.sync_copy(
            acc_sh.at[pl.ds(sid * RPS, RPS)],
            out_hbm.at[pl.ds(cid * N + sid * RPS, RPS)],
        )

    return k(cols2d, rows2d, vals2d, b_fc)


def kernel(a_indices, a_values, b, W, bias):
    pad = NNZ_PAD - NNZ
    rows = jnp.pad(a_indices[:, 0], (0, pad)).reshape(-1, CH)
    cols = jnp.pad(a_indices[:, 1], (0, pad)).reshape(-1, CH)
    vals = jnp.pad(a_values, (0, pad)).reshape(-1, CH)
    b_fc = _matmul(b, W, bias.reshape(1, D_OUT))
    partials = _sc_spmm(cols, rows, vals, b_fc)
    return _combine(partials)


# speedup vs baseline: 1.0517x; 1.0517x over previous
import functools

import jax
import jax.numpy as jnp
from jax import lax
from jax.experimental import pallas as pl
from jax.experimental.pallas import tpu as pltpu
from jax.experimental.pallas import tpu_sc as plsc

N = 16384
NNZ = 268435
D_IN = 1024
D_OUT = 32

NC = 2
NS = 16
NW = NC * NS
CH = 128
CHUNKS = -(-NNZ // (NW * CH))
PER_W = CHUNKS * CH
NNZ_PAD = NW * PER_W
RPS = N // NS


def _matmul(b, W, bias2d):
    m_blk = 2048

    def body(b_ref, w_ref, bias_ref, o_ref):
        o_ref[...] = (
            jnp.dot(b_ref[...], w_ref[...], preferred_element_type=jnp.float32)
            + bias_ref[...]
        )

    return pl.pallas_call(
        body,
        grid=(N // m_blk,),
        in_specs=[
            pl.BlockSpec((m_blk, D_IN), lambda i: (i, 0)),
            pl.BlockSpec((D_IN, D_OUT), lambda i: (0, 0)),
            pl.BlockSpec((1, D_OUT), lambda i: (0, 0)),
        ],
        out_specs=pl.BlockSpec((m_blk, D_OUT), lambda i: (i, 0)),
        out_shape=jax.ShapeDtypeStruct((N, D_OUT), jnp.float32),
    )(b, W, bias2d)


RW = N // NW


def _combine(partials):
    mesh = plsc.VectorSubcoreMesh(core_axis_name="c", subcore_axis_name="s")

    @functools.partial(
        pl.kernel,
        out_type=jax.ShapeDtypeStruct((N, D_OUT), jnp.float32),
        mesh=mesh,
        compiler_params=pltpu.CompilerParams(use_tc_tiling_on_sc=False),
        scratch_types=[
            pltpu.VMEM((RW, D_OUT), jnp.float32),
            pltpu.VMEM((RW, D_OUT), jnp.float32),
            pltpu.SemaphoreType.DMA,
            pltpu.SemaphoreType.DMA,
        ],
    )
    def k(p_hbm, out_hbm, t0, t1, s0, s1):
        cid = lax.axis_index("c")
        sid = lax.axis_index("s")
        base = (cid * NS + sid) * RW
        c0 = pltpu.async_copy(p_hbm.at[pl.ds(base, RW)], t0, s0)
        c1 = pltpu.async_copy(p_hbm.at[pl.ds(N + base, RW)], t1, s1)
        c0.wait()
        c1.wait()

        def add(i, _):
            t0[i, pl.ds(0, 16)] = t0[i, pl.ds(0, 16)] + t1[i, pl.ds(0, 16)]
            t0[i, pl.ds(16, 16)] = t0[i, pl.ds(16, 16)] + t1[i, pl.ds(16, 16)]
            return 0

        lax.fori_loop(0, RW, add, 0, unroll=4)
        pltpu.sync_copy(t0, out_hbm.at[pl.ds(base, RW)])

    return k(partials)


NB = 3


def _sc_spmm(cols2d, rows2d, vals2d, b_fc):
    mesh = plsc.VectorSubcoreMesh(core_axis_name="c", subcore_axis_name="s")

    @functools.partial(
        pl.kernel,
        out_type=jax.ShapeDtypeStruct((NC * N, D_OUT), jnp.float32),
        mesh=mesh,
        compiler_params=pltpu.CompilerParams(use_tc_tiling_on_sc=False),
        scratch_types=[
            pltpu.VMEM((CHUNKS, CH), jnp.int32),
            pltpu.VMEM((CHUNKS, CH), jnp.int32),
            pltpu.VMEM((CHUNKS, CH), jnp.float32),
            pltpu.VMEM((CH, D_OUT), jnp.float32),
            pltpu.VMEM((CH, D_OUT), jnp.float32),
            pltpu.VMEM((CH, D_OUT), jnp.float32),
            pltpu.VMEM_SHARED((N, D_OUT), jnp.float32),
            pltpu.VMEM_SHARED((N, D_OUT), jnp.float32),
            pltpu.SemaphoreType.DMA,
            pltpu.SemaphoreType.DMA,
            pltpu.SemaphoreType.DMA,
            pltpu.SemaphoreType.DMA,
            pltpu.SemaphoreType.DMA,
            pltpu.SemaphoreType.DMA,
            pltpu.SemaphoreType.DMA,
            pltpu.SemaphoreType.DMA,
            pltpu.SemaphoreType.DMA,
            pltpu.SemaphoreType.DMA,
        ],
    )
    def k(cols_hbm, rows_hbm, vals_hbm, bfc_hbm, out_hbm, col_all, row_all,
          val_all, g0, g1, g2, acc_sh, bfc_sh, sem_c, sem_r, sem_v, sem_b,
          sg0, sg1, sg2, ss0, ss1, ss2):
        gat = (g0, g1, g2)
        semg = (sg0, sg1, sg2)
        sems = (ss0, ss1, ss2)
        cid = lax.axis_index("c")
        sid = lax.axis_index("s")
        wid = cid * NS + sid

        c_copy = pltpu.async_copy(
            cols_hbm.at[pl.ds(wid * CHUNKS, CHUNKS)], col_all, sem_c)
        r_copy = pltpu.async_copy(
            rows_hbm.at[pl.ds(wid * CHUNKS, CHUNKS)], row_all, sem_r)
        v_copy = pltpu.async_copy(
            vals_hbm.at[pl.ds(wid * CHUNKS, CHUNKS)], val_all, sem_v)
        b_copy = pltpu.async_copy(
            bfc_hbm.at[pl.ds(sid * RPS, RPS)],
            bfc_sh.at[pl.ds(sid * RPS, RPS)], sem_b)

        def zfill(i, _):
            g0[i, pl.ds(0, 16)] = jnp.zeros((16,), jnp.float32)
            g0[i, pl.ds(16, 16)] = jnp.zeros((16,), jnp.float32)
            return 0

        lax.fori_loop(0, CH, zfill, 0, unroll=4)

        def zcopy(t, _):
            off = pl.multiple_of(sid * RPS + t * CH, CH)
            pltpu.sync_copy(g0, acc_sh.at[pl.ds(off, CH)])
            return 0

        lax.fori_loop(0, RPS // CH, zcopy, 0)
        c_copy.wait()
        r_copy.wait()
        v_copy.wait()
        b_copy.wait()
        plsc.subcore_barrier()

        for b in range(NB - 1):
            pltpu.async_copy(bfc_sh.at[col_all.at[b]], gat[b], semg[b])

        def scale(gbuf, t, j0):
            v16 = val_all[t, pl.ds(j0 * 16, 16)]
            for r in range(16):
                j = j0 * 16 + r
                gbuf[j, pl.ds(0, 16)] = gbuf[j, pl.ds(0, 16)] * v16[r]
                gbuf[j, pl.ds(16, 16)] = gbuf[j, pl.ds(16, 16)] * v16[r]

        def outer(g, _):
            for b in range(NB):
                t = g * NB + b
                pltpu.make_async_copy(
                    bfc_sh.at[col_all.at[t]], gat[b], semg[b]).wait()
                u = t + NB - 1
                bb = (b + NB - 1) % NB

                @pl.when(u < CHUNKS)
                def _():
                    @pl.when(t >= 1)
                    def _():
                        pltpu.make_async_copy(
                            gat[bb], acc_sh.at[row_all.at[t - 1]],
                            sems[bb]).wait()

                    pltpu.async_copy(
                        bfc_sh.at[col_all.at[u]], gat[bb], semg[bb])

                def sbody(j0, _):
                    scale(gat[b], t, j0)
                    return 0

                lax.fori_loop(0, CH // 16, sbody, 0)
                pltpu.async_copy(
                    gat[b], acc_sh.at[row_all.at[t]], sems[b], add=True)
            return 0

        lax.fori_loop(0, CHUNKS // NB, outer, 0)

        for b in range(NB):
            pltpu.make_async_copy(
                gat[b], acc_sh.at[row_all.at[CHUNKS - NB + b]],
                sems[b]).wait()
        plsc.subcore_barrier()

        pltpu.sync_copy(
            acc_sh.at[pl.ds(sid * RPS, RPS)],
            out_hbm.at[pl.ds(cid * N + sid * RPS, RPS)],
        )

    return k(cols2d, rows2d, vals2d, b_fc)


def kernel(a_indices, a_values, b, W, bias):
    pad = NNZ_PAD - NNZ
    rows = jnp.pad(a_indices[:, 0], (0, pad)).reshape(-1, CH)
    cols = jnp.pad(a_indices[:, 1], (0, pad)).reshape(-1, CH)
    vals = jnp.pad(a_values, (0, pad)).reshape(-1, CH)
    b_fc = _matmul(b, W, bias.reshape(1, D_OUT))
    partials = _sc_spmm(cols, rows, vals, b_fc)
    return _combine(partials)


# speedup vs baseline: 1.0519x; 1.0002x over previous
import functools

import jax
import jax.numpy as jnp
from jax import lax
from jax.experimental import pallas as pl
from jax.experimental.pallas import tpu as pltpu
from jax.experimental.pallas import tpu_sc as plsc

N = 16384
NNZ = 268435
D_IN = 1024
D_OUT = 32

NC = 2
NS = 16
NW = NC * NS
CH = 128
CHUNKS = -(-NNZ // (NW * CH))
PER_W = CHUNKS * CH
NNZ_PAD = NW * PER_W
RPS = N // NS


def _matmul(b, W, bias2d):
    m_blk = 2048

    def body(b_ref, w_ref, bias_ref, o_ref):
        o_ref[...] = (
            jnp.dot(b_ref[...], w_ref[...], preferred_element_type=jnp.float32)
            + bias_ref[...]
        )

    return pl.pallas_call(
        body,
        grid=(N // m_blk,),
        in_specs=[
            pl.BlockSpec((m_blk, D_IN), lambda i: (i, 0)),
            pl.BlockSpec((D_IN, D_OUT), lambda i: (0, 0)),
            pl.BlockSpec((1, D_OUT), lambda i: (0, 0)),
        ],
        out_specs=pl.BlockSpec((m_blk, D_OUT), lambda i: (i, 0)),
        out_shape=jax.ShapeDtypeStruct((N, D_OUT), jnp.float32),
    )(b, W, bias2d)


RW = N // NW


def _combine(partials):
    mesh = plsc.VectorSubcoreMesh(core_axis_name="c", subcore_axis_name="s")

    @functools.partial(
        pl.kernel,
        out_type=jax.ShapeDtypeStruct((N, D_OUT), jnp.float32),
        mesh=mesh,
        compiler_params=pltpu.CompilerParams(use_tc_tiling_on_sc=False),
        scratch_types=[
            pltpu.VMEM((RW, D_OUT), jnp.float32),
            pltpu.VMEM((RW, D_OUT), jnp.float32),
            pltpu.SemaphoreType.DMA,
            pltpu.SemaphoreType.DMA,
        ],
    )
    def k(p_hbm, out_hbm, t0, t1, s0, s1):
        cid = lax.axis_index("c")
        sid = lax.axis_index("s")
        base = (cid * NS + sid) * RW
        c0 = pltpu.async_copy(p_hbm.at[pl.ds(base, RW)], t0, s0)
        c1 = pltpu.async_copy(p_hbm.at[pl.ds(N + base, RW)], t1, s1)
        c0.wait()
        c1.wait()

        def add(i, _):
            t0[i, pl.ds(0, 16)] = t0[i, pl.ds(0, 16)] + t1[i, pl.ds(0, 16)]
            t0[i, pl.ds(16, 16)] = t0[i, pl.ds(16, 16)] + t1[i, pl.ds(16, 16)]
            return 0

        lax.fori_loop(0, RW, add, 0, unroll=4)
        pltpu.sync_copy(t0, out_hbm.at[pl.ds(base, RW)])

    return k(partials)


NB = 3


VALID_LAST = NNZ - (NW - 1) * PER_W
FULL_LAST = (VALID_LAST // CH) * CH
REM_LAST = VALID_LAST - FULL_LAST
ZSTART = (VALID_LAST // 16) * 16
ZCNT = PER_W - ZSTART


def _sc_spmm(cols1d, rows1d, vals1d, b_fc):
    mesh = plsc.VectorSubcoreMesh(core_axis_name="c", subcore_axis_name="s")

    @functools.partial(
        pl.kernel,
        out_type=jax.ShapeDtypeStruct((NC * N, D_OUT), jnp.float32),
        mesh=mesh,
        compiler_params=pltpu.CompilerParams(use_tc_tiling_on_sc=False),
        scratch_types=[
            pltpu.VMEM((PER_W,), jnp.int32),
            pltpu.VMEM((PER_W,), jnp.int32),
            pltpu.VMEM((PER_W,), jnp.float32),
            pltpu.VMEM((CH, D_OUT), jnp.float32),
            pltpu.VMEM((CH, D_OUT), jnp.float32),
            pltpu.VMEM((CH, D_OUT), jnp.float32),
            pltpu.VMEM_SHARED((N, D_OUT), jnp.float32),
            pltpu.VMEM_SHARED((N, D_OUT), jnp.float32),
            pltpu.SemaphoreType.DMA,
            pltpu.SemaphoreType.DMA,
            pltpu.SemaphoreType.DMA,
            pltpu.SemaphoreType.DMA,
            pltpu.SemaphoreType.DMA,
            pltpu.SemaphoreType.DMA,
            pltpu.SemaphoreType.DMA,
            pltpu.SemaphoreType.DMA,
            pltpu.SemaphoreType.DMA,
            pltpu.SemaphoreType.DMA,
        ],
    )
    def k(cols_hbm, rows_hbm, vals_hbm, bfc_hbm, out_hbm, col_all, row_all,
          val_all, g0, g1, g2, acc_sh, bfc_sh, sem_c, sem_r, sem_v, sem_b,
          sg0, sg1, sg2, ss0, ss1, ss2):
        gat = (g0, g1, g2)
        semg = (sg0, sg1, sg2)
        sems = (ss0, ss1, ss2)
        cid = lax.axis_index("c")
        sid = lax.axis_index("s")
        wid = cid * NS + sid

        @pl.when(wid < NW - 1)
        def _():
            pltpu.async_copy(
                cols_hbm.at[pl.ds(wid * PER_W, PER_W)], col_all, sem_c)
            pltpu.async_copy(
                rows_hbm.at[pl.ds(wid * PER_W, PER_W)], row_all, sem_r)
            pltpu.async_copy(
                vals_hbm.at[pl.ds(wid * PER_W, PER_W)], val_all, sem_v)

        @pl.when(wid == NW - 1)
        def _():
            zi = jnp.zeros((16,), jnp.int32)
            zf = jnp.zeros((16,), jnp.float32)

            def ztail(i, _):
                off = ZSTART + i * 16
                col_all[pl.ds(off, 16)] = zi
                row_all[pl.ds(off, 16)] = zi
                val_all[pl.ds(off, 16)] = zf
                return 0

            lax.fori_loop(0, ZCNT // 16, ztail, 0, unroll=4)
            base = wid * PER_W
            pltpu.async_copy(
                cols_hbm.at[pl.ds(base, FULL_LAST)],
                col_all.at[pl.ds(0, FULL_LAST)], sem_c)
            pltpu.async_copy(
                rows_hbm.at[pl.ds(base, FULL_LAST)],
                row_all.at[pl.ds(0, FULL_LAST)], sem_r)
            pltpu.async_copy(
                vals_hbm.at[pl.ds(base, FULL_LAST)],
                val_all.at[pl.ds(0, FULL_LAST)], sem_v)
            pltpu.async_copy(
                cols_hbm.at[pl.ds(base + FULL_LAST, REM_LAST)],
                col_all.at[pl.ds(FULL_LAST, REM_LAST)], sem_c)
            pltpu.async_copy(
                rows_hbm.at[pl.ds(base + FULL_LAST, REM_LAST)],
                row_all.at[pl.ds(FULL_LAST, REM_LAST)], sem_r)
            pltpu.async_copy(
                vals_hbm.at[pl.ds(base + FULL_LAST, REM_LAST)],
                val_all.at[pl.ds(FULL_LAST, REM_LAST)], sem_v)

        b_copy = pltpu.async_copy(
            bfc_hbm.at[pl.ds(sid * RPS, RPS)],
            bfc_sh.at[pl.ds(sid * RPS, RPS)], sem_b)

        def zfill(i, _):
            g0[i, pl.ds(0, 16)] = jnp.zeros((16,), jnp.float32)
            g0[i, pl.ds(16, 16)] = jnp.zeros((16,), jnp.float32)
            return 0

        lax.fori_loop(0, CH, zfill, 0, unroll=4)

        def zcopy(t, _):
            off = pl.multiple_of(sid * RPS + t * CH, CH)
            pltpu.sync_copy(g0, acc_sh.at[pl.ds(off, CH)])
            return 0

        lax.fori_loop(0, RPS // CH, zcopy, 0)

        @pl.when(wid < NW - 1)
        def _():
            pltpu.make_async_copy(
                cols_hbm.at[pl.ds(wid * PER_W, PER_W)], col_all, sem_c).wait()
            pltpu.make_async_copy(
                rows_hbm.at[pl.ds(wid * PER_W, PER_W)], row_all, sem_r).wait()
            pltpu.make_async_copy(
                vals_hbm.at[pl.ds(wid * PER_W, PER_W)], val_all, sem_v).wait()

        @pl.when(wid == NW - 1)
        def _():
            base = wid * PER_W
            for (hbm, dst, sem) in (
                (cols_hbm, col_all, sem_c),
                (rows_hbm, row_all, sem_r),
                (vals_hbm, val_all, sem_v),
            ):
                pltpu.make_async_copy(
                    hbm.at[pl.ds(base, FULL_LAST)],
                    dst.at[pl.ds(0, FULL_LAST)], sem).wait()
                pltpu.make_async_copy(
                    hbm.at[pl.ds(base + FULL_LAST, REM_LAST)],
                    dst.at[pl.ds(FULL_LAST, REM_LAST)], sem).wait()

        b_copy.wait()
        plsc.subcore_barrier()

        for b in range(NB - 1):
            pltpu.async_copy(
                bfc_sh.at[col_all.at[pl.ds(b * CH, CH)]], gat[b], semg[b])

        def scale(gbuf, t, j0):
            v16 = val_all[pl.ds(t * CH + j0 * 16, 16)]
            for r in range(16):
                j = j0 * 16 + r
                gbuf[j, pl.ds(0, 16)] = gbuf[j, pl.ds(0, 16)] * v16[r]
                gbuf[j, pl.ds(16, 16)] = gbuf[j, pl.ds(16, 16)] * v16[r]

        def outer(g, _):
            for b in range(NB):
                t = g * NB + b
                pltpu.make_async_copy(
                    bfc_sh.at[col_all.at[pl.ds(t * CH, CH)]],
                    gat[b], semg[b]).wait()
                u = t + NB - 1
                bb = (b + NB - 1) % NB

                @pl.when(u < CHUNKS)
                def _():
                    @pl.when(t >= 1)
                    def _():
                        pltpu.make_async_copy(
                            gat[bb],
                            acc_sh.at[row_all.at[pl.ds((t - 1) * CH, CH)]],
                            sems[bb]).wait()

                    pltpu.async_copy(
                        bfc_sh.at[col_all.at[pl.ds(u * CH, CH)]],
                        gat[bb], semg[bb])

                def sbody(j0, _):
                    scale(gat[b], t, j0)
                    return 0

                lax.fori_loop(0, CH // 16, sbody, 0)
                pltpu.async_copy(
                    gat[b], acc_sh.at[row_all.at[pl.ds(t * CH, CH)]],
                    sems[b], add=True)
            return 0

        lax.fori_loop(0, CHUNKS // NB, outer, 0)

        for b in range(NB):
            pltpu.make_async_copy(
                gat[b],
                acc_sh.at[row_all.at[pl.ds((CHUNKS - NB + b) * CH, CH)]],
                sems[b]).wait()
        plsc.subcore_barrier()

        pltpu.sync_copy(
            acc_sh.at[pl.ds(sid * RPS, RPS)],
            out_hbm.at[pl.ds(cid * N + sid * RPS, RPS)],
        )

    return k(cols1d, rows1d, vals1d, b_fc)


def kernel(a_indices, a_values, b, W, bias):
    b_fc = _matmul(b, W, bias.reshape(1, D_OUT))
    partials = _sc_spmm(a_indices[:, 1], a_indices[:, 0], a_values, b_fc)
    return _combine(partials)
